# baseline (speedup 1.0000x reference)
import functools

import jax
import jax.numpy as jnp
from jax import lax
from jax.experimental import pallas as pl
from jax.experimental.pallas import tpu as pltpu
from jax.experimental.pallas import tpu_sc as plsc

F32 = jnp.float32


def _emb_gather_body(idx_hbm, emb_hbm, out_hbm, idx_v, rows_v, sem, *, bpw):
    nc = plsc.get_sparse_core_info().num_cores
    wid = lax.axis_index("s") * nc + lax.axis_index("c")
    base = wid * bpw
    pltpu.sync_copy(idx_hbm.at[pl.ds(base, bpw)], idx_v)
    pltpu.async_copy(emb_hbm.at[idx_v], rows_v, sem).wait()
    pltpu.sync_copy(rows_v, out_hbm.at[pl.ds(base, bpw)])


def _emb_gather(idx, emb):
    T = idx.shape[0]
    C = emb.shape[1]
    info = plsc.get_sparse_core_info()
    nw = info.num_cores * info.num_subcores
    bpw = T // nw
    mesh = plsc.VectorSubcoreMesh(core_axis_name="c", subcore_axis_name="s")
    k = pl.kernel(
        functools.partial(_emb_gather_body, bpw=bpw),
        mesh=mesh,
        out_type=jax.ShapeDtypeStruct((T, C), F32),
        scratch_types=[
            pltpu.VMEM((bpw,), jnp.int32),
            pltpu.VMEM((bpw, C), F32),
            pltpu.SemaphoreType.DMA,
        ],
    )
    return k(idx, emb)


def _ln(x, g, b):
    m = jnp.mean(x, axis=-1, keepdims=True)
    v = jnp.mean((x - m) ** 2, axis=-1, keepdims=True)
    return (x - m) / jnp.sqrt(v + 1e-5) * g + b


def _tm_body(x_ref, g_ref, b_ref, wr_ref, wk_ref, wv_ref, wo_ref, o_ref,
             carry, scan, *, bt):
    i = pl.program_id(0)

    @pl.when(i == 0)
    def _():
        carry[...] = jnp.zeros_like(carry)

    xb = x_ref[...]
    h1 = _ln(xb, g_ref[...], b_ref[...])
    prev = carry[1:2, :]
    xs = jnp.concatenate([prev, h1[:-1, :]], axis=0)
    xm = 0.5 * (h1 + xs)
    r = jax.nn.sigmoid(jnp.dot(xm, wr_ref[...], preferred_element_type=F32))
    k = jnp.dot(xm, wk_ref[...], preferred_element_type=F32)
    v = jnp.dot(xm, wv_ref[...], preferred_element_type=F32)
    kv = k * v
    C = kv.shape[1]
    nsub = bt // 128
    scan[...] = kv.reshape(nsub, 128, C)

    def _step(rr, _):
        scan[:, pl.ds(rr, 1), :] += scan[:, pl.ds(rr - 1, 1), :]
        return 0

    lax.fori_loop(1, 128, _step, 0)
    cs3 = scan[...]
    c0 = carry[0:1, :]
    parts = []
    for s in range(nsub):
        parts.append(cs3[s] + c0)
        c0 = c0 + cs3[s, 127:128, :]
    carry[0:1, :] = c0
    cs = jnp.concatenate(parts, axis=0)
    carry[1:2, :] = h1[bt - 1:bt, :]
    denom = (i * bt + lax.broadcasted_iota(jnp.int32, (bt, 1), 0) + 1).astype(F32)
    att = jnp.dot(r * (cs / denom), wo_ref[...], preferred_element_type=F32)
    o_ref[...] = xb + att


def _time_mix(x, g, b, wr, wk, wv, wo, bt=256):
    T, C = x.shape
    grid = (T // bt,)
    return pl.pallas_call(
        functools.partial(_tm_body, bt=bt),
        grid=grid,
        in_specs=[
            pl.BlockSpec((bt, C), lambda i: (i, 0)),
            pl.BlockSpec((1, C), lambda i: (0, 0)),
            pl.BlockSpec((1, C), lambda i: (0, 0)),
            pl.BlockSpec((C, C), lambda i: (0, 0)),
            pl.BlockSpec((C, C), lambda i: (0, 0)),
            pl.BlockSpec((C, C), lambda i: (0, 0)),
            pl.BlockSpec((C, C), lambda i: (0, 0)),
        ],
        out_specs=pl.BlockSpec((bt, C), lambda i: (i, 0)),
        out_shape=jax.ShapeDtypeStruct((T, C), F32),
        scratch_shapes=[pltpu.VMEM((8, C), F32),
                        pltpu.VMEM((bt // 128, 128, C), F32)],
    )(x, g, b, wr, wk, wv, wo)


def _router_body(x_ref, g_ref, b_ref, cwt_ref, dw_ref, aw_ref, sh_ref,
                 h_ref, w_ref, s_ref):
    xb = x_ref[...]
    h = _ln(xb, g_ref[...], b_ref[...])
    confm = jax.nn.sigmoid(jnp.dot(h, cwt_ref[...], preferred_element_type=F32))
    diff = jax.nn.softplus(jnp.dot(h, dw_ref[...], preferred_element_type=F32))
    al = jnp.dot(h, aw_ref[...], preferred_element_type=F32)
    conf = [confm[:, e:e + 1] for e in range(3)]
    afl = [al[:, e:e + 1] for e in range(3)]
    m = jnp.maximum(jnp.maximum(afl[0], afl[1]), afl[2])
    ex = [jnp.exp(a - m) for a in afl]
    es = ex[0] + ex[1] + ex[2]
    sh = sh_ref[...]
    bids = [conf[e] * sh[:, e:e + 1] / (diff + 1.0) + ex[e] / es
            for e in range(3)]
    w01 = jnp.where(bids[0] >= bids[1], 0, 1)
    b01 = jnp.maximum(bids[0], bids[1])
    w = jnp.where(b01 >= bids[2], w01, 2).astype(jnp.int32)
    wb = jnp.where(w == 0, conf[0], jnp.where(w == 1, conf[1], conf[2]))
    s_ref[...] = wb / (wb + 1e-6)
    w_ref[...] = w
    h_ref[...] = h


def _router(x, g, b, cw, dw, awt, sh, bt=256):
    T, C = x.shape
    grid = (T // bt,)
    return pl.pallas_call(
        _router_body,
        grid=grid,
        in_specs=[
            pl.BlockSpec((bt, C), lambda i: (i, 0)),
            pl.BlockSpec((1, C), lambda i: (0, 0)),
            pl.BlockSpec((1, C), lambda i: (0, 0)),
            pl.BlockSpec((C, 3), lambda i: (0, 0)),
            pl.BlockSpec((C, 1), lambda i: (0, 0)),
            pl.BlockSpec((C, 3), lambda i: (0, 0)),
            pl.BlockSpec((1, 3), lambda i: (0, 0)),
        ],
        out_specs=[
            pl.BlockSpec((bt, C), lambda i: (i, 0)),
            pl.BlockSpec((bt, 1), lambda i: (i, 0)),
            pl.BlockSpec((bt, 1), lambda i: (i, 0)),
        ],
        out_shape=[
            jax.ShapeDtypeStruct((T, C), F32),
            jax.ShapeDtypeStruct((T, 1), jnp.int32),
            jax.ShapeDtypeStruct((T, 1), F32),
        ],
    )(x, g, b, cw, dw, awt, sh)


def _ffn_body(x_ref, h_ref, w1_ref, w2_ref, win_ref, sc_ref, o_ref, *, bt):
    e = pl.program_id(0)
    t = pl.program_id(1)
    rows = pl.ds(t * bt, bt)
    h = h_ref[...]
    z = jax.nn.gelu(jnp.dot(h, w1_ref[0], preferred_element_type=F32))
    z = z.astype(jnp.bfloat16).astype(F32)
    part = jnp.dot(z, w2_ref[0], preferred_element_type=F32)
    contrib = (win_ref[...] == e).astype(F32) * sc_ref[...] * part

    @pl.when(e == 0)
    def _():
        o_ref[rows, :] = x_ref[...] + contrib

    @pl.when(e != 0)
    def _():
        o_ref[rows, :] += contrib


def _moe_ffn(x, h, w1, w2, winners, scale, bt=512):
    T, C = h.shape
    E, _, H = w1.shape
    nt = T // bt
    return pl.pallas_call(
        functools.partial(_ffn_body, bt=bt),
        grid=(E, nt),
        in_specs=[
            pl.BlockSpec((bt, C), lambda e, t: (t, 0)),
            pl.BlockSpec((bt, C), lambda e, t: (t, 0)),
            pl.BlockSpec((1, C, H), lambda e, t: (e, 0, 0)),
            pl.BlockSpec((1, H, C), lambda e, t: (e, 0, 0)),
            pl.BlockSpec((bt, 1), lambda e, t: (t, 0)),
            pl.BlockSpec((bt, 1), lambda e, t: (t, 0)),
        ],
        out_specs=pl.BlockSpec((T, C), lambda e, t: (0, 0)),
        out_shape=jax.ShapeDtypeStruct((T, C), F32),
    )(x, h, w1, w2, winners, scale)


def _head_body(x_ref, g_ref, b_ref, w_ref, o_ref):
    h = _ln(x_ref[...], g_ref[...], b_ref[...])
    o_ref[...] = jnp.dot(h, w_ref[...], preferred_element_type=F32)


def _head(x, g, b, w, bv=3200, bt=512):
    T, C = x.shape
    V = w.shape[1]
    return pl.pallas_call(
        _head_body,
        grid=(V // bv, T // bt),
        in_specs=[
            pl.BlockSpec((bt, C), lambda v, t: (t, 0)),
            pl.BlockSpec((1, C), lambda v, t: (0, 0)),
            pl.BlockSpec((1, C), lambda v, t: (0, 0)),
            pl.BlockSpec((C, bv), lambda v, t: (0, v)),
        ],
        out_specs=pl.BlockSpec((bt, bv), lambda v, t: (t, v)),
        out_shape=jax.ShapeDtypeStruct((T, V), F32),
    )(x, g, b, w)


def _add_body(a_ref, b_ref, o_ref):
    o_ref[...] = a_ref[...] + b_ref[...]


def _add(a, b):
    return pl.pallas_call(_add_body, out_shape=jax.ShapeDtypeStruct(a.shape, F32))(a, b)


def kernel(idx, emb, ln1_g, ln1_b, ln2_g, ln2_b, Wr, Wk, Wv, Wo, W1, W2,
           conf_w, diff_w, aff_w, ln_out_g, ln_out_b, head_w, shares):
    B, T = idx.shape
    C = emb.shape[1]
    L = Wr.shape[0]

    x = _emb_gather(idx.reshape(T).astype(jnp.int32), emb)

    for i in range(L):
        x = _time_mix(x, ln1_g[i].reshape(1, C), ln1_b[i].reshape(1, C),
                      Wr[i], Wk[i], Wv[i], Wo[i])
        h, winners, scale = _router(
            x, ln2_g[i].reshape(1, C), ln2_b[i].reshape(1, C),
            jnp.transpose(conf_w[i]), diff_w[i].reshape(C, 1),
            aff_w[i], shares[i].reshape(1, 3))
        x = _moe_ffn(x, h, W1[i], W2[i], winners, scale)

    logits = _head(x, ln_out_g.reshape(1, C), ln_out_b.reshape(1, C), head_w)
    return logits.reshape(B, T, -1)
